# baseline (speedup 1.0000x reference)
import functools

import jax
import jax.numpy as jnp
from jax import lax
from jax.experimental import pallas as pl
from jax.experimental.pallas import tpu as pltpu
from jax.experimental.pallas import tpu_sc as plsc

_VOCAB = 90000
_EFFECT_DIM = 758
_EMBED_DIM = 768
_ADD_DIM = 10
_CTX = 1024

_NC = 2
_NS = 16
_NW = _NC * _NS
_BPW = _CTX // _NW
_NU = 96


def _sc_gather(ids, emb_table):
    mesh = plsc.VectorSubcoreMesh(core_axis_name="c", subcore_axis_name="s")
    D = _EFFECT_DIM

    @functools.partial(
        pl.kernel,
        mesh=mesh,
        out_type=jax.ShapeDtypeStruct((_CTX * D,), jnp.float32),
        scratch_types=[
            pltpu.VMEM((_BPW,), jnp.int32),
            pltpu.VMEM((_BPW * _NU,), jnp.int32),
            pltpu.VMEM((_BPW * _NU, 8), jnp.float32),
            pltpu.VMEM((_BPW * D,), jnp.float32),
            pltpu.SemaphoreType.DMA,
        ],
        compiler_params=pltpu.CompilerParams(use_tc_tiling_on_sc=False,
                                             needs_layout_passes=False),
    )
    def gather_k(ids_hbm, table8_hbm, out_hbm, idx_v, units_v, buf_v, al_v,
                 sem):
        wid = lax.axis_index("s") * _NC + lax.axis_index("c")
        base = wid * _BPW
        iota16 = lax.iota(jnp.int32, 16)
        pltpu.sync_copy(ids_hbm.at[pl.ds(base, _BPW)], idx_v)

        def row_id(r):
            half = idx_v[pl.ds((r // 16) * 16, 16)]
            return jnp.sum(jnp.where(iota16 == r % 16, half, 0))

        def build(r, carry):
            a8 = (row_id(r) * D) >> 3
            for c in range(_NU // 16):
                units_v[pl.ds(r * _NU + c * 16, 16)] = a8 + c * 16 + iota16
            return carry

        lax.fori_loop(0, _BPW, build, 0)
        pltpu.async_copy(table8_hbm.at[units_v], buf_v, sem).wait()

        def realign(r, carry):
            s = (row_id(r) * D) & 7
            off = r * (_NU * 8) + s
            for c in range(47):
                pos = off + c * 16 + iota16
                al_v[pl.ds(r * D + c * 16, 16)] = plsc.load_gather(
                    buf_v, [pos >> 3, pos & 7])
            pos = off + (D - 16) + iota16
            al_v[pl.ds(r * D + (D - 16), 16)] = plsc.load_gather(
                buf_v, [pos >> 3, pos & 7])
            return carry

        lax.fori_loop(0, _BPW, realign, 0)
        pltpu.sync_copy(al_v, out_hbm.at[pl.ds(base * D, _BPW * D)])

    table8 = emb_table.reshape(_VOCAB * D // 8, 8)
    return gather_k(ids, table8).reshape(_CTX, D)


def _mm_body(g_ref, w_ref, o_ref):
    o_ref[...] = jnp.dot(g_ref[...], w_ref[...],
                         preferred_element_type=jnp.float32)


def _tc_matmul(g, rev_weight, bn=1024):
    n_blocks = (_VOCAB + bn - 1) // bn
    return pl.pallas_call(
        _mm_body,
        grid=(n_blocks,),
        in_specs=[
            pl.BlockSpec((_CTX, _EMBED_DIM), lambda j: (0, 0)),
            pl.BlockSpec((_EMBED_DIM, bn), lambda j: (0, j)),
        ],
        out_specs=pl.BlockSpec((_CTX, bn), lambda j: (0, j)),
        out_shape=jax.ShapeDtypeStruct((_CTX, _VOCAB), jnp.float32),
    )(g, rev_weight)


def kernel(ids, emb_table, rev_weight):
    g = _sc_gather(ids, emb_table)
    pad = jnp.full((_CTX, _ADD_DIM), 0.1, dtype=jnp.float32)
    g_full = jnp.concatenate([g, pad], axis=1)
    return _tc_matmul(g_full, rev_weight)


# speedup vs baseline: 2.0946x; 2.0946x over previous
import functools

import jax
import jax.numpy as jnp
from jax import lax
from jax.experimental import pallas as pl
from jax.experimental.pallas import tpu as pltpu
from jax.experimental.pallas import tpu_sc as plsc

_VOCAB = 90000
_EFFECT_DIM = 758
_EMBED_DIM = 768
_ADD_DIM = 10
_CTX = 1024

_NC = 2
_NS = 16
_NW = _NC * _NS
_BPW = _CTX // _NW


def _sc_gather(ids, emb_table):
    mesh = plsc.VectorSubcoreMesh(core_axis_name="c", subcore_axis_name="s")

    @functools.partial(
        pl.kernel,
        mesh=mesh,
        out_type=jax.ShapeDtypeStruct((_CTX, _EFFECT_DIM), jnp.float32),
        scratch_types=[
            pltpu.VMEM((_BPW,), jnp.int32),
            pltpu.SemaphoreType.DMA,
        ],
        compiler_params=pltpu.CompilerParams(needs_layout_passes=False),
    )
    def gather_k(ids_hbm, table_hbm, out_hbm, idx_v, sem):
        wid = lax.axis_index("s") * _NC + lax.axis_index("c")
        base = wid * _BPW
        iota16 = lax.iota(jnp.int32, 16)
        pltpu.sync_copy(ids_hbm.at[pl.ds(base, _BPW)], idx_v)
        copies = []
        for r in range(_BPW):
            half = idx_v[pl.ds((r // 16) * 16, 16)]
            rid = jnp.sum(jnp.where(iota16 == r % 16, half, 0))
            copies.append(
                pltpu.async_copy(table_hbm.at[rid], out_hbm.at[base + r],
                                 sem))
        for cp in copies:
            cp.wait()

    return gather_k(ids, emb_table)


def _mm_body(g_ref, w_ref, o_ref):
    o_ref[...] = jnp.dot(g_ref[...], w_ref[...],
                         preferred_element_type=jnp.float32)


def _tc_matmul(g, rev_weight, bn=1024):
    n_blocks = (_VOCAB + bn - 1) // bn
    return pl.pallas_call(
        _mm_body,
        grid=(n_blocks,),
        in_specs=[
            pl.BlockSpec((_CTX, _EMBED_DIM), lambda j: (0, 0)),
            pl.BlockSpec((_EMBED_DIM, bn), lambda j: (0, j)),
        ],
        out_specs=pl.BlockSpec((_CTX, bn), lambda j: (0, j)),
        out_shape=jax.ShapeDtypeStruct((_CTX, _VOCAB), jnp.float32),
    )(g, rev_weight)


def kernel(ids, emb_table, rev_weight):
    g = _sc_gather(ids, emb_table)
    pad = jnp.full((_CTX, _ADD_DIM), 0.1, dtype=jnp.float32)
    g_full = jnp.concatenate([g, pad], axis=1)
    return _tc_matmul(g_full, rev_weight)


# speedup vs baseline: 2.1050x; 1.0050x over previous
import functools

import jax
import jax.numpy as jnp
from jax import lax
from jax.experimental import pallas as pl
from jax.experimental.pallas import tpu as pltpu
from jax.experimental.pallas import tpu_sc as plsc

_VOCAB = 90000
_EFFECT_DIM = 758
_EMBED_DIM = 768
_ADD_DIM = 10
_CTX = 1024

_NC = 2
_NS = 16
_NW = _NC * _NS
_BPW = _CTX // _NW


def _sc_gather(ids, emb_table):
    mesh = plsc.VectorSubcoreMesh(core_axis_name="c", subcore_axis_name="s")

    @functools.partial(
        pl.kernel,
        mesh=mesh,
        out_type=jax.ShapeDtypeStruct((_CTX, _EFFECT_DIM), jnp.float32),
        scratch_types=[
            pltpu.VMEM((_BPW,), jnp.int32),
            pltpu.SemaphoreType.DMA,
        ],
        compiler_params=pltpu.CompilerParams(needs_layout_passes=False),
    )
    def gather_k(ids_hbm, table_hbm, out_hbm, idx_v, sem):
        wid = lax.axis_index("s") * _NC + lax.axis_index("c")
        base = wid * _BPW
        iota16 = lax.iota(jnp.int32, 16)
        pltpu.sync_copy(ids_hbm.at[pl.ds(base, _BPW)], idx_v)
        copies = []
        for r in range(_BPW):
            half = idx_v[pl.ds((r // 16) * 16, 16)]
            rid = jnp.sum(jnp.where(iota16 == r % 16, half, 0))
            copies.append(
                pltpu.async_copy(table_hbm.at[rid], out_hbm.at[base + r],
                                 sem))
        for cp in copies:
            cp.wait()

    return gather_k(ids, emb_table)


def _mm_body(g_ref, w_ref, o_ref):
    o_ref[...] = jnp.dot(g_ref[...].astype(jnp.bfloat16),
                         w_ref[...].astype(jnp.bfloat16),
                         preferred_element_type=jnp.float32)


def _tc_matmul(g, rev_weight, bn=1024):
    n_blocks = (_VOCAB + bn - 1) // bn
    return pl.pallas_call(
        _mm_body,
        grid=(n_blocks,),
        in_specs=[
            pl.BlockSpec((_CTX, _EMBED_DIM), lambda j: (0, 0)),
            pl.BlockSpec((_EMBED_DIM, bn), lambda j: (0, j)),
        ],
        out_specs=pl.BlockSpec((_CTX, bn), lambda j: (0, j)),
        out_shape=jax.ShapeDtypeStruct((_CTX, _VOCAB), jnp.float32),
    )(g, rev_weight)


def kernel(ids, emb_table, rev_weight):
    g = _sc_gather(ids, emb_table)
    pad = jnp.full((_CTX, _ADD_DIM), 0.1, dtype=jnp.float32)
    g_full = jnp.concatenate([g, pad], axis=1)
    return _tc_matmul(g_full, rev_weight)
